# baseline (speedup 1.0000x reference)
import jax
import jax.numpy as jnp
from jax.experimental import pallas as pl
from jax.experimental.pallas import tpu as pltpu

_VOCAB_BLOCK = 3584
_BN = 2000
_NBUF = 3


def _row_scales(temp_ref, gid_ref):
    g = gid_ref[...]
    s = jnp.zeros(g.shape, dtype=jnp.float32)
    for gid in range(temp_ref.shape[0]):
        s = jnp.where(g == gid, 1.0 / temp_ref[gid], s)
    return s


def _make_manual_body(steps, bn):
    def body(temp_ref, gid_ref, x_hbm, o_hbm, ibuf, obuf, isem, osem):
        s = _row_scales(temp_ref, gid_ref)

        def in_copy(step):
            slot = jax.lax.rem(step, _NBUF)
            return pltpu.make_async_copy(
                x_hbm.at[pl.ds(step * bn, bn), :], ibuf.at[slot], isem.at[slot]
            )

        def out_copy(step):
            slot = jax.lax.rem(step, _NBUF)
            return pltpu.make_async_copy(
                obuf.at[slot], o_hbm.at[pl.ds(step * bn, bn), :], osem.at[slot]
            )

        for k in range(_NBUF):
            in_copy(jnp.int32(k)).start()

        def loop(step, carry):
            slot = jax.lax.rem(step, _NBUF)
            in_copy(step).wait()

            @pl.when(step >= _NBUF)
            def _():
                out_copy(step - _NBUF).wait()

            obuf[slot] = ibuf[slot] * s
            out_copy(step).start()

            @pl.when(step + _NBUF < steps)
            def _():
                in_copy(step + _NBUF).start()

            return carry

        jax.lax.fori_loop(jnp.int32(0), jnp.int32(steps), loop, 0)
        for k in range(min(_NBUF, steps)):
            out_copy(jnp.int32(steps - 1 - k)).wait()

    return body


def _auto_kernel(temp_ref, gid_ref, x_ref, o_ref):
    o_ref[...] = x_ref[...] * _row_scales(temp_ref, gid_ref)


def kernel(logits, group_ids, temperatures):
    batch, vocab = logits.shape
    xt = logits.T
    gid2 = group_ids.reshape(1, batch)

    if vocab % _BN == 0 and batch % 128 == 0:
        out_t = pl.pallas_call(
            _make_manual_body(vocab // _BN, _BN),
            in_specs=[
                pl.BlockSpec(memory_space=pltpu.SMEM),
                pl.BlockSpec((1, batch), lambda: (0, 0)),
                pl.BlockSpec(memory_space=pl.ANY),
            ],
            out_specs=pl.BlockSpec(memory_space=pl.ANY),
            out_shape=jax.ShapeDtypeStruct((vocab, batch), logits.dtype),
            scratch_shapes=[
                pltpu.VMEM((_NBUF, _BN, batch), jnp.float32),
                pltpu.VMEM((_NBUF, _BN, batch), jnp.float32),
                pltpu.SemaphoreType.DMA((_NBUF,)),
                pltpu.SemaphoreType.DMA((_NBUF,)),
            ],
        )(temperatures, gid2, xt)
    else:
        bn = _VOCAB_BLOCK
        out_t = pl.pallas_call(
            _auto_kernel,
            grid=(pl.cdiv(vocab, bn),),
            in_specs=[
                pl.BlockSpec(memory_space=pltpu.SMEM),
                pl.BlockSpec((1, batch), lambda j: (0, 0)),
                pl.BlockSpec((bn, batch), lambda j: (j, 0)),
            ],
            out_specs=pl.BlockSpec((bn, batch), lambda j: (j, 0)),
            out_shape=jax.ShapeDtypeStruct((vocab, batch), logits.dtype),
        )(temperatures, gid2, xt)
    return out_t.T


# speedup vs baseline: 1.0035x; 1.0035x over previous
import jax
import jax.numpy as jnp
from jax.experimental import pallas as pl
from jax.experimental.pallas import tpu as pltpu

_VOCAB_BLOCK = 3584
_BN = 2000
_NBUF = 3


def _row_scales(temp_ref, gid_ref):
    g = gid_ref[...]
    s = jnp.zeros(g.shape, dtype=jnp.float32)
    for gid in range(temp_ref.shape[0]):
        s = jnp.where(g == gid, 1.0 / temp_ref[gid], s)
    return s


def _make_manual_body(steps, bn):
    def body(temp_ref, gid_ref, x_hbm, o_hbm, ibuf, obuf, isem, osem):
        s = _row_scales(temp_ref, gid_ref)

        def in_copy(step):
            slot = jax.lax.rem(step, _NBUF)
            return pltpu.make_async_copy(
                x_hbm.at[pl.ds(step * bn, bn), :], ibuf.at[slot], isem.at[slot]
            )

        def out_copy(step):
            slot = jax.lax.rem(step, _NBUF)
            return pltpu.make_async_copy(
                obuf.at[slot], o_hbm.at[pl.ds(step * bn, bn), :], osem.at[slot]
            )

        for k in range(_NBUF):
            in_copy(jnp.int32(k)).start()

        def loop(step, carry):
            slot = jax.lax.rem(step, _NBUF)
            in_copy(step).wait()

            @pl.when(step >= _NBUF)
            def _():
                out_copy(step - _NBUF).wait()

            obuf[slot] = ibuf[slot] * s
            out_copy(step).start()

            @pl.when(step + _NBUF < steps)
            def _():
                in_copy(step + _NBUF).start()

            return carry

        jax.lax.fori_loop(jnp.int32(0), jnp.int32(steps), loop, 0)
        for k in range(min(_NBUF, steps)):
            out_copy(jnp.int32(steps - 1 - k)).wait()

    return body


def _auto_kernel(temp_ref, gid_ref, x_ref, o_ref):
    o_ref[...] = x_ref[...] * _row_scales(temp_ref, gid_ref)


def kernel(logits, group_ids, temperatures):
    batch, vocab = logits.shape
    xt = logits.T
    gid2 = group_ids.reshape(1, batch)

    if False and vocab % _BN == 0 and batch % 128 == 0:
        out_t = pl.pallas_call(
            _make_manual_body(vocab // _BN, _BN),
            in_specs=[
                pl.BlockSpec(memory_space=pltpu.SMEM),
                pl.BlockSpec((1, batch), lambda: (0, 0)),
                pl.BlockSpec(memory_space=pl.ANY),
            ],
            out_specs=pl.BlockSpec(memory_space=pl.ANY),
            out_shape=jax.ShapeDtypeStruct((vocab, batch), logits.dtype),
            scratch_shapes=[
                pltpu.VMEM((_NBUF, _BN, batch), jnp.float32),
                pltpu.VMEM((_NBUF, _BN, batch), jnp.float32),
                pltpu.SemaphoreType.DMA((_NBUF,)),
                pltpu.SemaphoreType.DMA((_NBUF,)),
            ],
        )(temperatures, gid2, xt)
    else:
        bn = _VOCAB_BLOCK
        out_t = pl.pallas_call(
            _auto_kernel,
            grid=(pl.cdiv(vocab, bn),),
            in_specs=[
                pl.BlockSpec(memory_space=pltpu.SMEM),
                pl.BlockSpec((1, batch), lambda j: (0, 0)),
                pl.BlockSpec((bn, batch), lambda j: (j, 0)),
            ],
            out_specs=pl.BlockSpec((bn, batch), lambda j: (j, 0)),
            out_shape=jax.ShapeDtypeStruct((vocab, batch), logits.dtype),
        )(temperatures, gid2, xt)
    return out_t.T
